# baseline (speedup 1.0000x reference)
import math

import numpy as np

import jax
import jax.numpy as jnp
from jax.experimental import pallas as pl
from jax.experimental.pallas import tpu as pltpu


def _simple_body(x_ref, w1t_ref, b1_ref, w2t_ref, b2_ref, o_ref):
    h = jax.lax.dot_general(
        x_ref[...], w1t_ref[...],
        dimension_numbers=(((1,), (0,)), ((), ())),
        preferred_element_type=jnp.float32,
    )
    h = jnp.maximum(h + b1_ref[...], 0.0)
    out = jax.lax.dot_general(
        h, w2t_ref[...],
        dimension_numbers=(((1,), (0,)), ((), ())),
        preferred_element_type=jnp.float32,
    )
    o_ref[...] = out + b2_ref[...]


def _simple_kernel(x, w1, b1, w2, b2):
    B, latent = x.shape
    H = w1.shape[0]
    C = w2.shape[0]
    tm = min(B, 8192)
    return pl.pallas_call(
        _simple_body,
        out_shape=jax.ShapeDtypeStruct((B, C), x.dtype),
        grid=(pl.cdiv(B, tm),),
        in_specs=[
            pl.BlockSpec((tm, latent), lambda i: (i, 0)),
            pl.BlockSpec((latent, H), lambda i: (0, 0)),
            pl.BlockSpec((1, H), lambda i: (0, 0)),
            pl.BlockSpec((H, C), lambda i: (0, 0)),
            pl.BlockSpec((1, C), lambda i: (0, 0)),
        ],
        out_specs=pl.BlockSpec((tm, C), lambda i: (i, 0)),
        compiler_params=pltpu.CompilerParams(
            dimension_semantics=("parallel",),
            vmem_limit_bytes=64 * 1024 * 1024,
        ),
    )(x, jnp.transpose(w1), b1.reshape(1, H), jnp.transpose(w2),
      b2.reshape(1, C))


def kernel(x, w1, b1, w2, b2):
    B, L = x.shape
    H = w1.shape[0]
    C = w2.shape[0]

    flat_per_group = (L * 128) // math.gcd(L, 128)
    P = flat_per_group // 128
    R = flat_per_group // L
    if B % R != 0:
        return _simple_kernel(x, w1, b1, w2, b2)
    G = B // R
    Gc = 1
    for d in (128, 64, 32, 16, 8, 4, 2):
        if G % d == 0:
            Gc = d
            break

    r0 = [(128 * p) // L for p in range(P)] + [R]
    m = [r0[p + 1] - r0[p] for p in range(P)]
    NS = max((128 * p + 127) // L - r0[p] + 1 for p in range(P))
    N1 = NS * H
    N2 = NS * C

    E3 = np.zeros((P * 128, NS, L), dtype=np.float32)
    for p in range(P):
        for l in range(128):
            q = 128 * p + l
            E3[q, q // L - r0[p], q % L] = 1.0

    w1t = jnp.transpose(w1)
    m_all = jnp.dot(jnp.asarray(E3.reshape(P * 128 * NS, L)), w1t)
    m_all = m_all.reshape(P * 128, NS * H)
    d_blk = jnp.kron(jnp.eye(NS, dtype=jnp.float32), jnp.transpose(w2))
    b1s = jnp.tile(b1, NS).reshape(1, N1)
    b2s = jnp.tile(b2, NS).reshape(1, N2)

    xg = x.reshape(G, P, 128)
    ooff = [C * r0[p] for p in range(P)]
    olen = [C * m[p] for p in range(P)]

    def body(xg_ref, m_ref, d_ref, b1_ref, b2_ref, o_ref):
        carry = None
        for p in range(P):
            xp = xg_ref[:, p, :]
            y = jax.lax.dot_general(
                xp, m_ref[pl.ds(128 * p, 128), :],
                dimension_numbers=(((1,), (0,)), ((), ())),
                preferred_element_type=jnp.float32,
            )
            if p > 0:
                y = jnp.concatenate([y[:, :H] + carry, y[:, H:]], axis=1)
            carry = y[:, H * m[p]:H * m[p] + H]
            h = jnp.maximum(y + b1_ref[...], 0.0)
            os_p = jax.lax.dot_general(
                h, d_ref[...],
                dimension_numbers=(((1,), (0,)), ((), ())),
                preferred_element_type=jnp.float32,
            ) + b2_ref[...]
            o_ref[:, ooff[p]:ooff[p] + olen[p]] = os_p[:, :olen[p]]

    od = pl.pallas_call(
        body,
        out_shape=jax.ShapeDtypeStruct((G, R * C), x.dtype),
        grid=(G // Gc,),
        in_specs=[
            pl.BlockSpec((Gc, P, 128), lambda i: (i, 0, 0)),
            pl.BlockSpec((P * 128, N1), lambda i: (0, 0)),
            pl.BlockSpec((N1, N2), lambda i: (0, 0)),
            pl.BlockSpec((1, N1), lambda i: (0, 0)),
            pl.BlockSpec((1, N2), lambda i: (0, 0)),
        ],
        out_specs=pl.BlockSpec((Gc, R * C), lambda i: (i, 0)),
        compiler_params=pltpu.CompilerParams(
            dimension_semantics=("parallel",),
            vmem_limit_bytes=64 * 1024 * 1024,
        ),
    )(xg, m_all, d_blk, b1s, b2s)
    return od.reshape(B, C)


# speedup vs baseline: 1.8981x; 1.8981x over previous
import jax
import jax.numpy as jnp
from jax.experimental import pallas as pl
from jax.experimental.pallas import tpu as pltpu

_NBUF = 8
_TS = 2048


def _simple_body(x_ref, w1t_ref, b1_ref, w2t_ref, b2_ref, o_ref):
    h = jax.lax.dot_general(
        x_ref[...], w1t_ref[...],
        dimension_numbers=(((1,), (0,)), ((), ())),
        preferred_element_type=jnp.float32,
    )
    h = jnp.maximum(h + b1_ref[...], 0.0)
    out = jax.lax.dot_general(
        h, w2t_ref[...],
        dimension_numbers=(((1,), (0,)), ((), ())),
        preferred_element_type=jnp.float32,
    )
    o_ref[...] = out + b2_ref[...]


def _simple_kernel(x, w1, b1, w2, b2):
    B, latent = x.shape
    H = w1.shape[0]
    C = w2.shape[0]
    tm = min(B, 8192)
    return pl.pallas_call(
        _simple_body,
        out_shape=jax.ShapeDtypeStruct((B, C), x.dtype),
        grid=(pl.cdiv(B, tm),),
        in_specs=[
            pl.BlockSpec((tm, latent), lambda i: (i, 0)),
            pl.BlockSpec((latent, H), lambda i: (0, 0)),
            pl.BlockSpec((1, H), lambda i: (0, 0)),
            pl.BlockSpec((H, C), lambda i: (0, 0)),
            pl.BlockSpec((1, C), lambda i: (0, 0)),
        ],
        out_specs=pl.BlockSpec((tm, C), lambda i: (i, 0)),
        compiler_params=pltpu.CompilerParams(
            dimension_semantics=("parallel",),
            vmem_limit_bytes=64 * 1024 * 1024,
        ),
    )(x, jnp.transpose(w1), b1.reshape(1, H), jnp.transpose(w2),
      b2.reshape(1, C))


def kernel(x, w1, b1, w2, b2):
    B, L = x.shape
    H = w1.shape[0]
    C = w2.shape[0]

    n_cores = 2
    if B % (n_cores * _TS) != 0 or (B // n_cores) // _TS < _NBUF:
        return _simple_kernel(x, w1, b1, w2, b2)
    rows_half = B // n_cores
    chunks = rows_half // _TS

    w1t = jnp.transpose(w1)
    w2t = jnp.transpose(w2)
    b1r = b1.reshape(1, H)
    b2r = b2.reshape(1, C)

    def body(x_hbm, w1t_r, b1_r, w2t_r, b2_r, o_hbm,
             xbuf, obuf, in_sems, out_sems):
        core = pl.program_id(0)
        base0 = core * rows_half

        def in_copy(c, slot):
            return pltpu.make_async_copy(
                x_hbm.at[pl.ds(base0 + c * _TS, _TS), :],
                xbuf.at[slot], in_sems.at[slot])

        def out_copy(c, slot):
            return pltpu.make_async_copy(
                obuf.at[slot],
                o_hbm.at[pl.ds(base0 + c * _TS, _TS), :], out_sems.at[slot])

        for s in range(_NBUF):
            in_copy(s, s).start()

        def step(i, carry):
            slot = jax.lax.rem(i, _NBUF)

            @pl.when(i >= _NBUF)
            def _():
                out_copy(i - _NBUF, slot).wait()

            in_copy(i, slot).wait()
            xr = xbuf[slot]
            h = jax.lax.dot_general(
                xr, w1t_r[...],
                dimension_numbers=(((1,), (0,)), ((), ())),
                preferred_element_type=jnp.float32,
            )
            h = jnp.maximum(h + b1_r[...], 0.0)
            o = jax.lax.dot_general(
                h, w2t_r[...],
                dimension_numbers=(((1,), (0,)), ((), ())),
                preferred_element_type=jnp.float32,
            )
            obuf[slot] = o + b2_r[...]
            out_copy(i, slot).start()

            @pl.when(i + _NBUF < chunks)
            def _():
                in_copy(i + _NBUF, slot).start()

            return carry

        jax.lax.fori_loop(0, chunks, step, 0)
        for s in range(_NBUF):
            out_copy(0, s).wait()

    return pl.pallas_call(
        body,
        out_shape=jax.ShapeDtypeStruct((B, C), x.dtype),
        grid=(n_cores,),
        in_specs=[
            pl.BlockSpec(memory_space=pl.ANY),
            pl.BlockSpec((L, H), lambda i: (0, 0)),
            pl.BlockSpec((1, H), lambda i: (0, 0)),
            pl.BlockSpec((H, C), lambda i: (0, 0)),
            pl.BlockSpec((1, C), lambda i: (0, 0)),
        ],
        out_specs=pl.BlockSpec(memory_space=pl.ANY),
        scratch_shapes=[
            pltpu.VMEM((_NBUF, _TS, L), jnp.float32),
            pltpu.VMEM((_NBUF, _TS, C), jnp.float32),
            pltpu.SemaphoreType.DMA((_NBUF,)),
            pltpu.SemaphoreType.DMA((_NBUF,)),
        ],
        compiler_params=pltpu.CompilerParams(
            dimension_semantics=("parallel",),
            vmem_limit_bytes=64 * 1024 * 1024,
        ),
    )(x, w1t, b1r, w2t, b2r)


# speedup vs baseline: 8.8016x; 4.6372x over previous
import jax
import jax.numpy as jnp
from jax.experimental import pallas as pl
from jax.experimental.pallas import tpu as pltpu

_TN = 8192
_SUB = 1024


def _body(xt_ref, w1_ref, b1_ref, w2_ref, b2_ref, o_ref):
    tn = xt_ref.shape[1]
    for start in range(0, tn, _SUB):
        sl = pl.ds(start, min(_SUB, tn - start))
        xc = xt_ref[:, sl]
        h = jax.lax.dot_general(
            w1_ref[...], xc,
            dimension_numbers=(((1,), (0,)), ((), ())),
            preferred_element_type=jnp.float32,
        )
        h = jnp.maximum(h + b1_ref[...], 0.0)
        o = jax.lax.dot_general(
            w2_ref[...], h,
            dimension_numbers=(((1,), (0,)), ((), ())),
            preferred_element_type=jnp.float32,
        )
        o_ref[:, sl] = o + b2_ref[...]


def kernel(x, w1, b1, w2, b2):
    B, L = x.shape
    H = w1.shape[0]
    C = w2.shape[0]

    xt = jnp.transpose(x)
    b1c = b1.reshape(H, 1)
    b2c = b2.reshape(C, 1)

    tn = _TN if B % _TN == 0 else B
    grid = (pl.cdiv(B, tn),)

    ot = pl.pallas_call(
        _body,
        out_shape=jax.ShapeDtypeStruct((C, B), x.dtype),
        grid=grid,
        in_specs=[
            pl.BlockSpec((L, tn), lambda i: (0, i)),
            pl.BlockSpec((H, L), lambda i: (0, 0)),
            pl.BlockSpec((H, 1), lambda i: (0, 0)),
            pl.BlockSpec((C, H), lambda i: (0, 0)),
            pl.BlockSpec((C, 1), lambda i: (0, 0)),
        ],
        out_specs=pl.BlockSpec((C, tn), lambda i: (0, i)),
        compiler_params=pltpu.CompilerParams(
            dimension_semantics=("parallel",),
            vmem_limit_bytes=64 * 1024 * 1024,
        ),
    )(xt, w1, b1c, w2, b2c)
    return jnp.transpose(ot)


# speedup vs baseline: 21.6334x; 2.4579x over previous
import jax
import jax.numpy as jnp
from jax.experimental import pallas as pl
from jax.experimental.pallas import tpu as pltpu

_TN = 32768
_SUB = 8192


def _body(xt_ref, w1_ref, b1_ref, w2_ref, b2_ref, o_ref):
    tn = xt_ref.shape[1]
    for start in range(0, tn, _SUB):
        sl = pl.ds(start, min(_SUB, tn - start))
        xc = xt_ref[:, sl]
        h = jax.lax.dot_general(
            w1_ref[...], xc,
            dimension_numbers=(((1,), (0,)), ((), ())),
            preferred_element_type=jnp.float32,
        )
        h = jnp.maximum(h + b1_ref[...], 0.0)
        o = jax.lax.dot_general(
            w2_ref[...], h,
            dimension_numbers=(((1,), (0,)), ((), ())),
            preferred_element_type=jnp.float32,
        )
        o_ref[:, sl] = o + b2_ref[...]


def kernel(x, w1, b1, w2, b2):
    B, L = x.shape
    H = w1.shape[0]
    C = w2.shape[0]

    xt = jnp.transpose(x)
    b1c = b1.reshape(H, 1)
    b2c = b2.reshape(C, 1)

    tn = _TN if B % _TN == 0 else B
    grid = (pl.cdiv(B, tn),)

    ot = pl.pallas_call(
        _body,
        out_shape=jax.ShapeDtypeStruct((C, B), x.dtype),
        grid=grid,
        in_specs=[
            pl.BlockSpec((L, tn), lambda i: (0, i)),
            pl.BlockSpec((H, L), lambda i: (0, 0)),
            pl.BlockSpec((H, 1), lambda i: (0, 0)),
            pl.BlockSpec((C, H), lambda i: (0, 0)),
            pl.BlockSpec((C, 1), lambda i: (0, 0)),
        ],
        out_specs=pl.BlockSpec((C, tn), lambda i: (0, i)),
        compiler_params=pltpu.CompilerParams(
            dimension_semantics=("parallel",),
            vmem_limit_bytes=64 * 1024 * 1024,
        ),
    )(xt, w1, b1c, w2, b2c)
    return jnp.transpose(ot)
